# Initial kernel scaffold; baseline (speedup 1.0000x reference)
import jax
import jax.numpy as jnp
from jax.experimental import pallas as pl


def kernel(layer_values_0, layer_values_1):
    raise NotImplementedError("write your pallas kernel here")


# speedup vs baseline: 1.5995x; 1.5995x over previous
import functools

import jax
import jax.numpy as jnp
import numpy as np
from jax import lax
from jax.experimental import pallas as pl
from jax.experimental.pallas import tpu as pltpu
from jax.experimental.pallas import tpu_sc as plsc

_PAIRS = [[0,3],[1,17],[0,250],[1,999],[0,1500],[1,4096],[0,7777],[1,12345],[0,20000],[1,33333],[0,45000],[1,54321],[0,60000],[1,77777],[0,88888],[1,99998],[1,3],[0,17],[1,250],[0,999],[1,1500],[0,4096],[1,7777],[0,12345],[1,20000],[0,33333],[1,45000],[0,54321],[1,60000],[0,77777],[1,88888],[0,99998],[0,3],[0,99998],[1,17],[1,88888],[0,250],[0,77777],[1,999],[1,60000],[0,1500],[0,54321],[1,4096],[1,45000],[0,7777],[0,33333],[1,12345],[1,20000]]

_N_OUT = len(_PAIRS)
_D = 128

_src_list = [[], []]
_dst_list = [[], []]
for _i, (_l, _o) in enumerate(_PAIRS):
    _src_list[_l].append(_o)
    _dst_list[_l].append(_i)
_N_PER = len(_src_list[0])
assert len(_src_list[1]) == _N_PER and _N_PER % 8 == 0

_SRC_IDX = np.asarray(_src_list, dtype=np.int32)
_DST_IDX = np.asarray(_dst_list, dtype=np.int32)

_info = plsc.get_sparse_core_info()
_NC = _info.num_cores


def _sc_body(t0_hbm, t1_hbm, src_hbm, dst_hbm, out_hbm,
             src_v, dst_v, rows_v, gsem, ssem):
    wid = lax.axis_index("s") * _NC + lax.axis_index("c")

    @pl.when(wid == 0)
    def _layer0():
        pltpu.sync_copy(src_hbm.at[0], src_v)
        pltpu.sync_copy(dst_hbm.at[0], dst_v)
        pltpu.async_copy(t0_hbm.at[src_v], rows_v, gsem).wait()
        pltpu.async_copy(rows_v, out_hbm.at[dst_v], ssem).wait()

    @pl.when(wid == 1)
    def _layer1():
        pltpu.sync_copy(src_hbm.at[1], src_v)
        pltpu.sync_copy(dst_hbm.at[1], dst_v)
        pltpu.async_copy(t1_hbm.at[src_v], rows_v, gsem).wait()
        pltpu.async_copy(rows_v, out_hbm.at[dst_v], ssem).wait()


_gather_call = functools.partial(
    pl.kernel,
    mesh=plsc.VectorSubcoreMesh(core_axis_name="c", subcore_axis_name="s"),
    out_type=jax.ShapeDtypeStruct((_N_OUT, _D), jnp.float32),
    scratch_types=[
        pltpu.VMEM((_N_PER,), jnp.int32),
        pltpu.VMEM((_N_PER,), jnp.int32),
        pltpu.VMEM((_N_PER, _D), jnp.float32),
        pltpu.SemaphoreType.DMA,
        pltpu.SemaphoreType.DMA,
    ],
)(_sc_body)


@jax.jit
def kernel(layer_values_0, layer_values_1):
    src = jnp.asarray(_SRC_IDX)
    dst = jnp.asarray(_DST_IDX)
    return _gather_call(layer_values_0, layer_values_1, src, dst)


# speedup vs baseline: 1.7587x; 1.0996x over previous
import functools

import jax
import jax.numpy as jnp
import numpy as np
from jax import lax
from jax.experimental import pallas as pl
from jax.experimental.pallas import tpu as pltpu
from jax.experimental.pallas import tpu_sc as plsc

_PAIRS = [[0,3],[1,17],[0,250],[1,999],[0,1500],[1,4096],[0,7777],[1,12345],[0,20000],[1,33333],[0,45000],[1,54321],[0,60000],[1,77777],[0,88888],[1,99998],[1,3],[0,17],[1,250],[0,999],[1,1500],[0,4096],[1,7777],[0,12345],[1,20000],[0,33333],[1,45000],[0,54321],[1,60000],[0,77777],[1,88888],[0,99998],[0,3],[0,99998],[1,17],[1,88888],[0,250],[0,77777],[1,999],[1,60000],[0,1500],[0,54321],[1,4096],[1,45000],[0,7777],[0,33333],[1,12345],[1,20000]]

_N_OUT = len(_PAIRS)
_D = 128

_src_list = [[], []]
_dst_list = [[], []]
for _i, (_l, _o) in enumerate(_PAIRS):
    _src_list[_l].append(_o)
    _dst_list[_l].append(_i)
_N_PER = len(_src_list[0])
assert len(_src_list[1]) == _N_PER and _N_PER % 8 == 0

_IDX = np.stack([np.asarray(_src_list, dtype=np.int32),
                 np.asarray(_dst_list, dtype=np.int32)], axis=1)


def _sc_body(t0_hbm, t1_hbm, idx_hbm, out_hbm, idx_v, rows_v, gsem, ssem):
    sid = lax.axis_index("s")

    @pl.when(sid == 0)
    def _layer0():
        pltpu.sync_copy(idx_hbm.at[0], idx_v)
        pltpu.async_copy(t0_hbm.at[idx_v.at[0]], rows_v, gsem).wait()
        pltpu.async_copy(rows_v, out_hbm.at[idx_v.at[1]], ssem).wait()

    @pl.when(sid == 1)
    def _layer1():
        pltpu.sync_copy(idx_hbm.at[1], idx_v)
        pltpu.async_copy(t1_hbm.at[idx_v.at[0]], rows_v, gsem).wait()
        pltpu.async_copy(rows_v, out_hbm.at[idx_v.at[1]], ssem).wait()


_gather_call = functools.partial(
    pl.kernel,
    mesh=plsc.VectorSubcoreMesh(core_axis_name="c", subcore_axis_name="s",
                                num_cores=1),
    out_type=jax.ShapeDtypeStruct((_N_OUT, _D), jnp.float32),
    scratch_types=[
        pltpu.VMEM((2, _N_PER), jnp.int32),
        pltpu.VMEM((_N_PER, _D), jnp.float32),
        pltpu.SemaphoreType.DMA,
        pltpu.SemaphoreType.DMA,
    ],
)(_sc_body)


@jax.jit
def kernel(layer_values_0, layer_values_1):
    return _gather_call(layer_values_0, layer_values_1, jnp.asarray(_IDX))


# speedup vs baseline: 1.7779x; 1.0109x over previous
import functools

import jax
import jax.numpy as jnp
import numpy as np
from jax import lax
from jax.experimental import pallas as pl
from jax.experimental.pallas import tpu as pltpu
from jax.experimental.pallas import tpu_sc as plsc

_PAIRS = [[0,3],[1,17],[0,250],[1,999],[0,1500],[1,4096],[0,7777],[1,12345],[0,20000],[1,33333],[0,45000],[1,54321],[0,60000],[1,77777],[0,88888],[1,99998],[1,3],[0,17],[1,250],[0,999],[1,1500],[0,4096],[1,7777],[0,12345],[1,20000],[0,33333],[1,45000],[0,54321],[1,60000],[0,77777],[1,88888],[0,99998],[0,3],[0,99998],[1,17],[1,88888],[0,250],[0,77777],[1,999],[1,60000],[0,1500],[0,54321],[1,4096],[1,45000],[0,7777],[0,33333],[1,12345],[1,20000]]

_N_OUT = len(_PAIRS)
_D = 128
_CH = 16

_src_list = [[], []]
_dst_list = [[], []]
for _i, (_l, _o) in enumerate(_PAIRS):
    _src_list[_l].append(_o)
    _dst_list[_l].append(_i)
assert len(_src_list[0]) == len(_src_list[1]) == 24

_CHUNKS = []
for _l in (0, 1):
    for _lo in (0, 8):
        _CHUNKS.append((
            _l,
            np.asarray(_src_list[_l][_lo:_lo + _CH], dtype=np.int32),
            np.asarray(_dst_list[_l][_lo:_lo + _CH], dtype=np.int32),
        ))


def _const_vec(vals):
    lane = lax.iota(jnp.int32, 16)
    acc = jnp.where(lane == 0, int(vals[0]), 0)
    for k in range(1, 16):
        acc = jnp.where(lane == k, int(vals[k]), acc)
    return acc


def _sc_body(t0_hbm, t1_hbm, out_hbm, rows_v, gsem, ssem):
    sid = lax.axis_index("s")
    tables = (t0_hbm, t1_hbm)

    for _t, (_layer, _src, _dst) in enumerate(_CHUNKS):
        @pl.when(sid == _t)
        def _(tbl=tables[_layer], src=_src, dst=_dst):
            src_reg = _const_vec(src)
            dst_reg = _const_vec(dst)
            pltpu.async_copy(tbl.at[src_reg], rows_v, gsem).wait()
            pltpu.async_copy(rows_v, out_hbm.at[dst_reg], ssem).wait()


_gather_call = functools.partial(
    pl.kernel,
    mesh=plsc.VectorSubcoreMesh(core_axis_name="c", subcore_axis_name="s",
                                num_cores=1),
    out_type=jax.ShapeDtypeStruct((_N_OUT, _D), jnp.float32),
    scratch_types=[
        pltpu.VMEM((_CH, _D), jnp.float32),
        pltpu.SemaphoreType.DMA,
        pltpu.SemaphoreType.DMA,
    ],
)(_sc_body)


@jax.jit
def kernel(layer_values_0, layer_values_1):
    return _gather_call(layer_values_0, layer_values_1)


# speedup vs baseline: 1.9231x; 1.0817x over previous
import functools

import jax
import jax.numpy as jnp
from jax import lax
from jax.experimental import pallas as pl
from jax.experimental.pallas import tpu as pltpu
from jax.experimental.pallas import tpu_sc as plsc

_PAIRS = [[0,3],[1,17],[0,250],[1,999],[0,1500],[1,4096],[0,7777],[1,12345],[0,20000],[1,33333],[0,45000],[1,54321],[0,60000],[1,77777],[0,88888],[1,99998],[1,3],[0,17],[1,250],[0,999],[1,1500],[0,4096],[1,7777],[0,12345],[1,20000],[0,33333],[1,45000],[0,54321],[1,60000],[0,77777],[1,88888],[0,99998],[0,3],[0,99998],[1,17],[1,88888],[0,250],[0,77777],[1,999],[1,60000],[0,1500],[0,54321],[1,4096],[1,45000],[0,7777],[0,33333],[1,12345],[1,20000]]

_N_OUT = len(_PAIRS)
_D = 128


def _sc_body(t0_hbm, t1_hbm, out_hbm, sem):
    tables = (t0_hbm, t1_hbm)
    copies = []
    for i, (layer, row) in enumerate(_PAIRS):
        copies.append(
            pltpu.async_copy(tables[layer].at[row], out_hbm.at[i], sem))
    for c in copies:
        c.wait()


_gather_call = functools.partial(
    pl.kernel,
    mesh=plsc.ScalarSubcoreMesh(axis_name="c", num_cores=1),
    out_type=jax.ShapeDtypeStruct((_N_OUT, _D), jnp.float32),
    scratch_types=[
        pltpu.SemaphoreType.DMA,
    ],
)(_sc_body)


@jax.jit
def kernel(layer_values_0, layer_values_1):
    return _gather_call(layer_values_0, layer_values_1)


# speedup vs baseline: 1.9398x; 1.0087x over previous
import functools

import jax
import jax.numpy as jnp
from jax import lax
from jax.experimental import pallas as pl
from jax.experimental.pallas import tpu as pltpu
from jax.experimental.pallas import tpu_sc as plsc

_PAIRS = [[0,3],[1,17],[0,250],[1,999],[0,1500],[1,4096],[0,7777],[1,12345],[0,20000],[1,33333],[0,45000],[1,54321],[0,60000],[1,77777],[0,88888],[1,99998],[1,3],[0,17],[1,250],[0,999],[1,1500],[0,4096],[1,7777],[0,12345],[1,20000],[0,33333],[1,45000],[0,54321],[1,60000],[0,77777],[1,88888],[0,99998],[0,3],[0,99998],[1,17],[1,88888],[0,250],[0,77777],[1,999],[1,60000],[0,1500],[0,54321],[1,4096],[1,45000],[0,7777],[0,33333],[1,12345],[1,20000]]

_N_OUT = len(_PAIRS)
_D = 128


def _sc_body(t0_hbm, t1_hbm, out_hbm, sem):
    tables = (t0_hbm, t1_hbm)
    for i, (layer, row) in enumerate(_PAIRS):
        pltpu.async_copy(tables[layer].at[row], out_hbm.at[i], sem)
    pltpu.make_async_copy(t0_hbm.at[pl.ds(0, _N_OUT)], out_hbm, sem).wait()


_gather_call = functools.partial(
    pl.kernel,
    mesh=plsc.ScalarSubcoreMesh(axis_name="c", num_cores=1),
    out_type=jax.ShapeDtypeStruct((_N_OUT, _D), jnp.float32),
    scratch_types=[
        pltpu.SemaphoreType.DMA,
    ],
)(_sc_body)


@jax.jit
def kernel(layer_values_0, layer_values_1):
    return _gather_call(layer_values_0, layer_values_1)
